# baseline (speedup 1.0000x reference)
import functools
import jax
import jax.numpy as jnp
from jax import lax
from jax.experimental import pallas as pl
from jax.experimental.pallas import tpu as pltpu
from jax.experimental.pallas import tpu_sc as plsc

_N = 10000
_C = 128
_HC = _C // 2
_E = 320000
_NC = 2
_NS = 16
_NW = _NC * _NS
_EPW = _E // _NW
_CH = 80
_NCHUNK = _EPW // _CH
_EPT = _E // _NS
_CH2 = 128
_NCHUNK2 = _EPT // _CH2
_L = 16
_EPS = 1e-6


def _prep_body(x_ref, wc_ref, wp_ref, wv_ref, ab_ref, phichi_ref, n2_ref):
    xb = x_ref[...]
    y = jnp.dot(xb, wc_ref[...].T, preferred_element_type=jnp.float32)
    a = jnp.dot(y, wp_ref[...].T, preferred_element_type=jnp.float32)
    b = jnp.dot(xb, wv_ref[...].T, preferred_element_type=jnp.float32)
    ab_ref[:, :_C] = a.astype(jnp.bfloat16)
    ab_ref[:, _C:] = b.astype(jnp.bfloat16)
    phichi_ref[...] = jnp.tanh(jnp.sqrt(jnp.sum(y * y, axis=1, keepdims=True)))
    n2_ref[...] = jnp.sum(b * b, axis=1, keepdims=True)


def _prep(x, W_chi, W_phi, W_varphi):
    n, c = x.shape
    blk = 1000
    return pl.pallas_call(
        _prep_body,
        grid=(n // blk,),
        in_specs=[
            pl.BlockSpec((blk, c), lambda i: (i, 0)),
            pl.BlockSpec((c, c), lambda i: (0, 0)),
            pl.BlockSpec((c, c), lambda i: (0, 0)),
            pl.BlockSpec((c, c), lambda i: (0, 0)),
        ],
        out_specs=[
            pl.BlockSpec((blk, 2 * c), lambda i: (i, 0)),
            pl.BlockSpec((blk, 1), lambda i: (i, 0)),
            pl.BlockSpec((blk, 1), lambda i: (i, 0)),
        ],
        out_shape=[
            jax.ShapeDtypeStruct((n, 2 * c), jnp.bfloat16),
            jax.ShapeDtypeStruct((n, 1), jnp.float32),
            jax.ShapeDtypeStruct((n, 1), jnp.float32),
        ],
    )(x, W_chi, W_phi, W_varphi)


def _tanh_pos(t):
    t = jnp.minimum(t, 15.0)
    return 1.0 - 2.0 / (jnp.exp(2.0 * t) + 1.0)


def _sqrt16(v):
    i = plsc.bitcast(v, jnp.int32)
    r = plsc.bitcast(jnp.int32(0x5F3759DF) - (i >> 1), jnp.float32)
    for _ in range(3):
        r = r * (1.5 - 0.5 * v * r * r)
    return v * r


def _sc_phi_body(row_h, col_h, ab_h, n2_h, zero1_h,
                 phi_out, s_out, deg_out,
                 rowv, colv, abr, abc, acc1, acc2, phib, onesb, n2v,
                 s_acc, deg_acc, sem):
    cid = lax.axis_index("c")
    sid = lax.axis_index("s")
    wid = cid * _NS + sid

    pltpu.sync_copy(n2_h, n2v)
    for g in range(_CH // _L):
        onesb[pl.ds(g * _L, _L)] = jnp.full((_L,), 1.0, jnp.float32)

    @pl.when(sid == 0)
    def _zero():
        pltpu.sync_copy(zero1_h, s_acc)
        pltpu.sync_copy(zero1_h, deg_acc)

    plsc.subcore_barrier()

    def chunk(k, carry):
        base = wid * _EPW + k * _CH
        pltpu.sync_copy(row_h.at[pl.ds(base, _CH)], rowv)
        pltpu.sync_copy(col_h.at[pl.ds(base, _CH)], colv)
        cp1 = pltpu.async_copy(ab_h.at[rowv], abr, sem)
        cp2 = pltpu.async_copy(ab_h.at[colv], abc, sem)
        cp1.wait()
        cp2.wait()

        def dot_e(e, c2):
            def dot_half(off):
                acc = None
                for j in range(_C // (2 * _L)):
                    ar = plsc.bitcast(abr[e, pl.ds(off + _L * j, _L)],
                                      jnp.bfloat16)
                    ac = plsc.bitcast(abc[e, pl.ds(off + _L * j, _L)],
                                      jnp.bfloat16)
                    u1, u2 = plsc.unpack(
                        ar, format=plsc.PackFormat.INTERLEAVED,
                        preferred_element_type=jnp.float32)
                    v1, v2 = plsc.unpack(
                        ac, format=plsc.PackFormat.INTERLEAVED,
                        preferred_element_type=jnp.float32)
                    term = u1 * v1 + u2 * v2
                    acc = term if acc is None else acc + term
                return acc

            acc1[pl.ds(e * _L, _L)] = dot_half(0)
            acc2[pl.ds(e * _L, _L)] = dot_half(_C // 2)
            return c2

        lax.fori_loop(0, _CH, dot_e, 0)

        for g in range(_CH // _L):
            fl = (lax.iota(jnp.int32, _L) + (g * _L)) * _L
            d1 = plsc.load_gather(acc1, [fl])
            d2 = plsc.load_gather(acc2, [fl])
            for j in range(1, _L):
                d1 = d1 + plsc.load_gather(acc1, [fl + j])
                d2 = d2 + plsc.load_gather(acc2, [fl + j])
            r16 = rowv[pl.ds(g * _L, _L)]
            c16 = colv[pl.ds(g * _L, _L)]
            n2r = plsc.load_gather(n2v, [r16])
            n2c = plsc.load_gather(n2v, [c16])
            nd2 = jnp.maximum(n2r + n2c - 2.0 * d2, 0.0)
            pv = _tanh_pos(1.0 / (_sqrt16(nd2) + _EPS))
            pp = _tanh_pos(jnp.abs(d1))
            phib[pl.ds(g * _L, _L)] = pp * pv

        pltpu.sync_copy(phib, phi_out.at[pl.ds(base, _CH)])
        pltpu.sync_copy(phib, s_acc.at[rowv], add=True)
        pltpu.sync_copy(onesb, deg_acc.at[rowv], add=True)
        return carry

    lax.fori_loop(0, _NCHUNK, chunk, 0)

    plsc.subcore_barrier()

    @pl.when(sid == 0)
    def _writeout():
        pltpu.sync_copy(s_acc, s_out.at[cid])
        pltpu.sync_copy(deg_acc, deg_out.at[cid])


_sc_phi = functools.partial(
    pl.kernel,
    mesh=plsc.VectorSubcoreMesh(core_axis_name="c", subcore_axis_name="s"),
    compiler_params=pltpu.CompilerParams(needs_layout_passes=False),
    out_type=[
        jax.ShapeDtypeStruct((_E,), jnp.float32),
        jax.ShapeDtypeStruct((_NC, _N), jnp.float32),
        jax.ShapeDtypeStruct((_NC, _N), jnp.float32),
    ],
    scratch_types=[
        pltpu.VMEM((_CH,), jnp.int32),
        pltpu.VMEM((_CH,), jnp.int32),
        pltpu.VMEM((_CH, _C), jnp.int32),
        pltpu.VMEM((_CH, _C), jnp.int32),
        pltpu.VMEM((_CH * _L,), jnp.float32),
        pltpu.VMEM((_CH * _L,), jnp.float32),
        pltpu.VMEM((_CH,), jnp.float32),
        pltpu.VMEM((_CH,), jnp.float32),
        pltpu.VMEM((_N,), jnp.float32),
        pltpu.VMEM_SHARED((_N,), jnp.float32),
        pltpu.VMEM_SHARED((_N,), jnp.float32),
        pltpu.SemaphoreType.DMA,
    ],
)(_sc_phi_body)


_HN = _N // _NC
_JUNK = _HN
_TROWS = _HN + 8


def _sc_t_body(row_h, col_h, phi_h, x_h, zero2_h,
               t_out,
               rowv, colv, adjv, phib, xc, t_acc, sem):
    cid = lax.axis_index("c")
    sid = lax.axis_index("s")

    @pl.when(sid == 0)
    def _zero():
        pltpu.sync_copy(zero2_h, t_acc)

    plsc.subcore_barrier()

    row_lo = cid * _HN

    def chunk(k, carry):
        base = sid * _EPT + k * _CH2
        pltpu.sync_copy(row_h.at[pl.ds(base, _CH2)], rowv)
        pltpu.sync_copy(col_h.at[pl.ds(base, _CH2)], colv)
        pltpu.sync_copy(phi_h.at[pl.ds(base, _CH2)], phib)
        pltpu.async_copy(x_h.at[colv], xc, sem).wait()

        for g in range(_CH2 // _L):
            rv = rowv[pl.ds(g * _L, _L)]
            adj = rv - row_lo
            ok = jnp.logical_and(adj >= 0, adj < _HN)
            adjv[pl.ds(g * _L, _L)] = jnp.where(ok, adj, _JUNK)

        def scale_e(e, c2):
            p = plsc.load_gather(phib, [jnp.full((_L,), e, jnp.int32)])
            for j in range(_C // _L):
                sl = pl.ds(_L * j, _L)
                xc[e, sl] = xc[e, sl] * p
            return c2

        lax.fori_loop(0, _CH2, scale_e, 0)

        pltpu.sync_copy(xc, t_acc.at[adjv], add=True)
        return carry

    lax.fori_loop(0, _NCHUNK2, chunk, 0)

    plsc.subcore_barrier()

    @pl.when(sid < 5)
    def _writeout():
        pltpu.sync_copy(t_acc.at[pl.ds(sid * 1000, 1000)],
                        t_out.at[cid, pl.ds(sid * 1000, 1000)])


_sc_t = functools.partial(
    pl.kernel,
    mesh=plsc.VectorSubcoreMesh(core_axis_name="c", subcore_axis_name="s"),
    compiler_params=pltpu.CompilerParams(needs_layout_passes=False),
    out_type=[
        jax.ShapeDtypeStruct((_NC, _HN, _C), jnp.float32),
    ],
    scratch_types=[
        pltpu.VMEM((_CH2,), jnp.int32),
        pltpu.VMEM((_CH2,), jnp.int32),
        pltpu.VMEM((_CH2,), jnp.int32),
        pltpu.VMEM((_CH2,), jnp.float32),
        pltpu.VMEM((_CH2, _C), jnp.float32),
        pltpu.VMEM_SHARED((_TROWS, _C), jnp.float32),
        pltpu.SemaphoreType.DMA,
    ],
)(_sc_t_body)


def _comb_body(x_ref, pc_ref, s_ref, dg_ref, t_ref, o_ref):
    xv = x_ref[...]
    s = s_ref[0] + s_ref[1]
    dg = dg_ref[0] + dg_ref[1]
    tt = t_ref[0]
    pc = pc_ref[...]
    scale = jnp.where(dg > 0, 0.5 / (dg * pc + 1e-30), 0.0)
    o_ref[...] = xv - scale * (s * xv - tt)


def _combine(x, phichi, s_parts, deg_parts, t_parts):
    n, c = x.shape
    blk = 1000
    return pl.pallas_call(
        _comb_body,
        grid=(n // blk,),
        in_specs=[
            pl.BlockSpec((blk, c), lambda i: (i, 0)),
            pl.BlockSpec((blk, 1), lambda i: (i, 0)),
            pl.BlockSpec((_NC, blk, 1), lambda i: (0, i, 0)),
            pl.BlockSpec((_NC, blk, 1), lambda i: (0, i, 0)),
            pl.BlockSpec((1, blk, _C), lambda i: (i // 5, i % 5, 0)),
        ],
        out_specs=pl.BlockSpec((blk, c), lambda i: (i, 0)),
        out_shape=jax.ShapeDtypeStruct((n, c), jnp.float32),
    )(x, phichi, s_parts, deg_parts, t_parts)


def kernel(x, edge_index, W_chi, W_phi, W_varphi):
    ab, phichi, n2 = _prep(x, W_chi, W_phi, W_varphi)
    row = edge_index[0]
    col = edge_index[1]
    zeros2 = jnp.zeros((_TROWS, _C), jnp.float32)
    zeros1 = jnp.zeros((_N,), jnp.float32)
    ab_i32 = jax.lax.bitcast_convert_type(
        ab.reshape(_N, _C, 2), jnp.int32)
    phi, s_parts, deg_parts = _sc_phi(row, col, ab_i32, n2[:, 0], zeros1)
    (t_parts,) = _sc_t(row, col, phi, x, zeros2)
    return _combine(x, phichi, s_parts[..., None], deg_parts[..., None],
                    t_parts)


# speedup vs baseline: 1.0065x; 1.0065x over previous
import functools
import jax
import jax.numpy as jnp
from jax import lax
from jax.experimental import pallas as pl
from jax.experimental.pallas import tpu as pltpu
from jax.experimental.pallas import tpu_sc as plsc

_N = 10000
_C = 128
_HC = _C // 2
_E = 320000
_NC = 2
_NS = 16
_NW = _NC * _NS
_EPW = _E // _NW
_CH = 80
_NCHUNK = _EPW // _CH
_EPT = _E // _NS
_CH2 = 128
_NCHUNK2 = _EPT // _CH2
_L = 16
_EPS = 1e-6


def _prep_body(x_ref, wc_ref, wp_ref, wv_ref, ab_ref, phichi_ref, n2_ref):
    xb = x_ref[...]
    y = jnp.dot(xb, wc_ref[...].T, preferred_element_type=jnp.float32)
    a = jnp.dot(y, wp_ref[...].T, preferred_element_type=jnp.float32)
    b = jnp.dot(xb, wv_ref[...].T, preferred_element_type=jnp.float32)
    ab_ref[:, :_C] = a.astype(jnp.bfloat16)
    ab_ref[:, _C:] = b.astype(jnp.bfloat16)
    phichi_ref[...] = jnp.tanh(jnp.sqrt(jnp.sum(y * y, axis=1, keepdims=True)))
    n2_ref[...] = jnp.sum(b * b, axis=1, keepdims=True)


def _prep(x, W_chi, W_phi, W_varphi):
    n, c = x.shape
    blk = 1000
    return pl.pallas_call(
        _prep_body,
        grid=(n // blk,),
        in_specs=[
            pl.BlockSpec((blk, c), lambda i: (i, 0)),
            pl.BlockSpec((c, c), lambda i: (0, 0)),
            pl.BlockSpec((c, c), lambda i: (0, 0)),
            pl.BlockSpec((c, c), lambda i: (0, 0)),
        ],
        out_specs=[
            pl.BlockSpec((blk, 2 * c), lambda i: (i, 0)),
            pl.BlockSpec((blk, 1), lambda i: (i, 0)),
            pl.BlockSpec((blk, 1), lambda i: (i, 0)),
        ],
        out_shape=[
            jax.ShapeDtypeStruct((n, 2 * c), jnp.bfloat16),
            jax.ShapeDtypeStruct((n, 1), jnp.float32),
            jax.ShapeDtypeStruct((n, 1), jnp.float32),
        ],
    )(x, W_chi, W_phi, W_varphi)


def _tanh_pos(t):
    t = jnp.minimum(t, 15.0)
    return 1.0 - 2.0 / (jnp.exp(2.0 * t) + 1.0)


def _sqrt16(v):
    i = plsc.bitcast(v, jnp.int32)
    r = plsc.bitcast(jnp.int32(0x5F3759DF) - (i >> 1), jnp.float32)
    for _ in range(3):
        r = r * (1.5 - 0.5 * v * r * r)
    return v * r


def _sc_phi_body(row_h, col_h, ab_h, n2_h, zero1_h,
                 phi_out, s_out, deg_out,
                 rowv, colv, abr, abc, acc1, acc2, phib, onesb, n2v,
                 s_acc, deg_acc, sem):
    cid = lax.axis_index("c")
    sid = lax.axis_index("s")
    wid = cid * _NS + sid

    pltpu.sync_copy(n2_h, n2v)
    for g in range(_CH // _L):
        onesb[pl.ds(g * _L, _L)] = jnp.full((_L,), 1.0, jnp.float32)

    @pl.when(sid == 0)
    def _zero():
        pltpu.sync_copy(zero1_h, s_acc)
        pltpu.sync_copy(zero1_h, deg_acc)

    plsc.subcore_barrier()

    def chunk(k, carry):
        base = wid * _EPW + k * _CH
        pltpu.sync_copy(row_h.at[pl.ds(base, _CH)], rowv)
        pltpu.sync_copy(col_h.at[pl.ds(base, _CH)], colv)
        cp1 = pltpu.async_copy(ab_h.at[rowv], abr, sem)
        cp2 = pltpu.async_copy(ab_h.at[colv], abc, sem)
        cp1.wait()
        cp2.wait()

        def dot_e(e, c2):
            def dot_half(off):
                acc = None
                for j in range(_C // (2 * _L)):
                    ar = plsc.bitcast(abr[e, pl.ds(off + _L * j, _L)],
                                      jnp.bfloat16)
                    ac = plsc.bitcast(abc[e, pl.ds(off + _L * j, _L)],
                                      jnp.bfloat16)
                    term = ar * ac
                    acc = term if acc is None else acc + term
                u1, u2 = plsc.unpack(
                    acc, format=plsc.PackFormat.INTERLEAVED,
                    preferred_element_type=jnp.float32)
                return u1 + u2

            acc1[pl.ds(e * _L, _L)] = dot_half(0)
            acc2[pl.ds(e * _L, _L)] = dot_half(_C // 2)
            return c2

        lax.fori_loop(0, _CH, dot_e, 0)

        for g in range(_CH // _L):
            fl = (lax.iota(jnp.int32, _L) + (g * _L)) * _L
            d1 = plsc.load_gather(acc1, [fl])
            d2 = plsc.load_gather(acc2, [fl])
            for j in range(1, _L):
                d1 = d1 + plsc.load_gather(acc1, [fl + j])
                d2 = d2 + plsc.load_gather(acc2, [fl + j])
            r16 = rowv[pl.ds(g * _L, _L)]
            c16 = colv[pl.ds(g * _L, _L)]
            n2r = plsc.load_gather(n2v, [r16])
            n2c = plsc.load_gather(n2v, [c16])
            nd2 = jnp.maximum(n2r + n2c - 2.0 * d2, 0.0)
            pv = _tanh_pos(1.0 / (_sqrt16(nd2) + _EPS))
            pp = _tanh_pos(jnp.abs(d1))
            phib[pl.ds(g * _L, _L)] = pp * pv

        pltpu.sync_copy(phib, phi_out.at[pl.ds(base, _CH)])
        pltpu.sync_copy(phib, s_acc.at[rowv], add=True)
        pltpu.sync_copy(onesb, deg_acc.at[rowv], add=True)
        return carry

    lax.fori_loop(0, _NCHUNK, chunk, 0)

    plsc.subcore_barrier()

    @pl.when(sid == 0)
    def _writeout():
        pltpu.sync_copy(s_acc, s_out.at[cid])
        pltpu.sync_copy(deg_acc, deg_out.at[cid])


_sc_phi = functools.partial(
    pl.kernel,
    mesh=plsc.VectorSubcoreMesh(core_axis_name="c", subcore_axis_name="s"),
    compiler_params=pltpu.CompilerParams(needs_layout_passes=False),
    out_type=[
        jax.ShapeDtypeStruct((_E,), jnp.float32),
        jax.ShapeDtypeStruct((_NC, _N), jnp.float32),
        jax.ShapeDtypeStruct((_NC, _N), jnp.float32),
    ],
    scratch_types=[
        pltpu.VMEM((_CH,), jnp.int32),
        pltpu.VMEM((_CH,), jnp.int32),
        pltpu.VMEM((_CH, _C), jnp.int32),
        pltpu.VMEM((_CH, _C), jnp.int32),
        pltpu.VMEM((_CH * _L,), jnp.float32),
        pltpu.VMEM((_CH * _L,), jnp.float32),
        pltpu.VMEM((_CH,), jnp.float32),
        pltpu.VMEM((_CH,), jnp.float32),
        pltpu.VMEM((_N,), jnp.float32),
        pltpu.VMEM_SHARED((_N,), jnp.float32),
        pltpu.VMEM_SHARED((_N,), jnp.float32),
        pltpu.SemaphoreType.DMA,
    ],
)(_sc_phi_body)


_HN = _N // _NC
_JUNK = _HN
_TROWS = _HN + 8


def _sc_t_body(row_h, col_h, phi_h, x_h, zero2_h,
               t_out,
               rowv, colv, adjv, phib, xc, t_acc, sem):
    cid = lax.axis_index("c")
    sid = lax.axis_index("s")

    @pl.when(sid == 0)
    def _zero():
        pltpu.sync_copy(zero2_h, t_acc)

    plsc.subcore_barrier()

    row_lo = cid * _HN

    def chunk(k, carry):
        base = sid * _EPT + k * _CH2
        pltpu.sync_copy(row_h.at[pl.ds(base, _CH2)], rowv)
        pltpu.sync_copy(col_h.at[pl.ds(base, _CH2)], colv)
        pltpu.sync_copy(phi_h.at[pl.ds(base, _CH2)], phib)
        pltpu.async_copy(x_h.at[colv], xc, sem).wait()

        for g in range(_CH2 // _L):
            rv = rowv[pl.ds(g * _L, _L)]
            adj = rv - row_lo
            ok = jnp.logical_and(adj >= 0, adj < _HN)
            adjv[pl.ds(g * _L, _L)] = jnp.where(ok, adj, _JUNK)

        def scale_e(e, c2):
            p = plsc.load_gather(phib, [jnp.full((_L,), e, jnp.int32)])
            for j in range(_C // _L):
                sl = pl.ds(_L * j, _L)
                xc[e, sl] = xc[e, sl] * p
            return c2

        lax.fori_loop(0, _CH2, scale_e, 0)

        pltpu.sync_copy(xc, t_acc.at[adjv], add=True)
        return carry

    lax.fori_loop(0, _NCHUNK2, chunk, 0)

    plsc.subcore_barrier()

    @pl.when(sid < 5)
    def _writeout():
        pltpu.sync_copy(t_acc.at[pl.ds(sid * 1000, 1000)],
                        t_out.at[cid, pl.ds(sid * 1000, 1000)])


_sc_t = functools.partial(
    pl.kernel,
    mesh=plsc.VectorSubcoreMesh(core_axis_name="c", subcore_axis_name="s"),
    compiler_params=pltpu.CompilerParams(needs_layout_passes=False),
    out_type=[
        jax.ShapeDtypeStruct((_NC, _HN, _C), jnp.float32),
    ],
    scratch_types=[
        pltpu.VMEM((_CH2,), jnp.int32),
        pltpu.VMEM((_CH2,), jnp.int32),
        pltpu.VMEM((_CH2,), jnp.int32),
        pltpu.VMEM((_CH2,), jnp.float32),
        pltpu.VMEM((_CH2, _C), jnp.float32),
        pltpu.VMEM_SHARED((_TROWS, _C), jnp.float32),
        pltpu.SemaphoreType.DMA,
    ],
)(_sc_t_body)


def _comb_body(x_ref, pc_ref, s_ref, dg_ref, t_ref, o_ref):
    xv = x_ref[...]
    s = s_ref[0] + s_ref[1]
    dg = dg_ref[0] + dg_ref[1]
    tt = t_ref[0]
    pc = pc_ref[...]
    scale = jnp.where(dg > 0, 0.5 / (dg * pc + 1e-30), 0.0)
    o_ref[...] = xv - scale * (s * xv - tt)


def _combine(x, phichi, s_parts, deg_parts, t_parts):
    n, c = x.shape
    blk = 1000
    return pl.pallas_call(
        _comb_body,
        grid=(n // blk,),
        in_specs=[
            pl.BlockSpec((blk, c), lambda i: (i, 0)),
            pl.BlockSpec((blk, 1), lambda i: (i, 0)),
            pl.BlockSpec((_NC, blk, 1), lambda i: (0, i, 0)),
            pl.BlockSpec((_NC, blk, 1), lambda i: (0, i, 0)),
            pl.BlockSpec((1, blk, _C), lambda i: (i // 5, i % 5, 0)),
        ],
        out_specs=pl.BlockSpec((blk, c), lambda i: (i, 0)),
        out_shape=jax.ShapeDtypeStruct((n, c), jnp.float32),
    )(x, phichi, s_parts, deg_parts, t_parts)


def kernel(x, edge_index, W_chi, W_phi, W_varphi):
    ab, phichi, n2 = _prep(x, W_chi, W_phi, W_varphi)
    row = edge_index[0]
    col = edge_index[1]
    zeros2 = jnp.zeros((_TROWS, _C), jnp.float32)
    zeros1 = jnp.zeros((_N,), jnp.float32)
    ab_i32 = jax.lax.bitcast_convert_type(
        ab.reshape(_N, _C, 2), jnp.int32)
    phi, s_parts, deg_parts = _sc_phi(row, col, ab_i32, n2[:, 0], zeros1)
    (t_parts,) = _sc_t(row, col, phi, x, zeros2)
    return _combine(x, phichi, s_parts[..., None], deg_parts[..., None],
                    t_parts)


# speedup vs baseline: 1.8432x; 1.8312x over previous
import functools
import jax
import jax.numpy as jnp
from jax import lax
from jax.experimental import pallas as pl
from jax.experimental.pallas import tpu as pltpu
from jax.experimental.pallas import tpu_sc as plsc

_N = 10000
_C = 128
_HC = _C // 2
_E = 320000
_NC = 2
_NS = 16
_NW = _NC * _NS
_EPW = _E // _NW
_CH = 80
_NCHUNK = _EPW // _CH
_EPT = _E // _NS
_CH2 = 80
_NCHUNK2 = _EPT // _CH2
_L = 16
_EPS = 1e-6


def _prep_body(x_ref, wc_ref, wp_ref, wv_ref, ab_ref, phichi_ref, n2_ref):
    xb = x_ref[...]
    y = jnp.dot(xb, wc_ref[...].T, preferred_element_type=jnp.float32)
    a = jnp.dot(y, wp_ref[...].T, preferred_element_type=jnp.float32)
    b = jnp.dot(xb, wv_ref[...].T, preferred_element_type=jnp.float32)
    ab_ref[:, :_C] = a.astype(jnp.bfloat16)
    ab_ref[:, _C:] = b.astype(jnp.bfloat16)
    phichi_ref[...] = jnp.tanh(jnp.sqrt(jnp.sum(y * y, axis=1, keepdims=True)))
    n2_ref[...] = jnp.sum(b * b, axis=1, keepdims=True)


def _prep(x, W_chi, W_phi, W_varphi):
    n, c = x.shape
    blk = 1000
    return pl.pallas_call(
        _prep_body,
        grid=(n // blk,),
        in_specs=[
            pl.BlockSpec((blk, c), lambda i: (i, 0)),
            pl.BlockSpec((c, c), lambda i: (0, 0)),
            pl.BlockSpec((c, c), lambda i: (0, 0)),
            pl.BlockSpec((c, c), lambda i: (0, 0)),
        ],
        out_specs=[
            pl.BlockSpec((blk, 2 * c), lambda i: (i, 0)),
            pl.BlockSpec((blk, 1), lambda i: (i, 0)),
            pl.BlockSpec((blk, 1), lambda i: (i, 0)),
        ],
        out_shape=[
            jax.ShapeDtypeStruct((n, 2 * c), jnp.bfloat16),
            jax.ShapeDtypeStruct((n, 1), jnp.float32),
            jax.ShapeDtypeStruct((n, 1), jnp.float32),
        ],
    )(x, W_chi, W_phi, W_varphi)


def _tanh_pos(t):
    t = jnp.minimum(t, 15.0)
    return 1.0 - 2.0 / (jnp.exp(2.0 * t) + 1.0)


def _sqrt16(v):
    i = plsc.bitcast(v, jnp.int32)
    r = plsc.bitcast(jnp.int32(0x5F3759DF) - (i >> 1), jnp.float32)
    for _ in range(3):
        r = r * (1.5 - 0.5 * v * r * r)
    return v * r


def _sc_phi_body(row_h, col_h, ab_h, n2_h, zero1_h,
                 phi_out, s_out, deg_out,
                 rowa, cola, abr, abc, acc1, acc2, phia, rows, onesb, n2v,
                 s_acc, deg_acc, gsem0, gsem1, ssem0, ssem1):
    cid = lax.axis_index("c")
    sid = lax.axis_index("s")
    wid = cid * _NS + sid
    ebase = wid * _EPW

    pltpu.sync_copy(n2_h, n2v)
    pltpu.sync_copy(row_h.at[pl.ds(ebase, _EPW)], rowa)
    pltpu.sync_copy(col_h.at[pl.ds(ebase, _EPW)], cola)
    for g in range(_CH // _L):
        onesb[pl.ds(g * _L, _L)] = jnp.full((_L,), 1.0, jnp.float32)

    @pl.when(sid == 0)
    def _zero():
        pltpu.sync_copy(zero1_h, s_acc)
        pltpu.sync_copy(zero1_h, deg_acc)

    plsc.subcore_barrier()

    gsems = (gsem0, gsem1)
    ssems = (ssem0, ssem1)

    def fire_gathers(k, b):
        sl = pl.ds(k * _CH, _CH)
        pltpu.async_copy(ab_h.at[rowa.at[sl]], abr.at[b], gsems[b])
        pltpu.async_copy(ab_h.at[cola.at[sl]], abc.at[b], gsems[b])

    def wait_gathers(b):
        sl = pl.ds(0, _CH)
        pltpu.make_async_copy(ab_h.at[rowa.at[sl]], abr.at[b],
                              gsems[b]).wait()
        pltpu.make_async_copy(ab_h.at[cola.at[sl]], abc.at[b],
                              gsems[b]).wait()

    def drain_scatters(b):
        pltpu.make_async_copy(phia.at[pl.ds(0, _CH)],
                              s_acc.at[rows.at[b]], ssems[b]).wait()
        pltpu.make_async_copy(onesb, deg_acc.at[rows.at[b]],
                              ssems[b]).wait()

    def compute(k, b):
        def dot_e(e, c2):
            def dot_half(off):
                acc = None
                for j in range(_C // (2 * _L)):
                    ar = plsc.bitcast(abr[b, e, pl.ds(off + _L * j, _L)],
                                      jnp.bfloat16)
                    ac = plsc.bitcast(abc[b, e, pl.ds(off + _L * j, _L)],
                                      jnp.bfloat16)
                    term = ar * ac
                    acc = term if acc is None else acc + term
                u1, u2 = plsc.unpack(
                    acc, format=plsc.PackFormat.INTERLEAVED,
                    preferred_element_type=jnp.float32)
                return u1 + u2

            acc1[pl.ds(e * _L, _L)] = dot_half(0)
            acc2[pl.ds(e * _L, _L)] = dot_half(_C // 2)
            return c2

        lax.fori_loop(0, _CH, dot_e, 0)

        for g in range(_CH // _L):
            fl = (lax.iota(jnp.int32, _L) + (g * _L)) * _L
            d1 = plsc.load_gather(acc1, [fl])
            d2 = plsc.load_gather(acc2, [fl])
            for j in range(1, _L):
                d1 = d1 + plsc.load_gather(acc1, [fl + j])
                d2 = d2 + plsc.load_gather(acc2, [fl + j])
            off = k * _CH + g * _L
            r16 = rowa[pl.ds(off, _L)]
            c16 = cola[pl.ds(off, _L)]
            rows[b, pl.ds(g * _L, _L)] = r16
            n2r = plsc.load_gather(n2v, [r16])
            n2c = plsc.load_gather(n2v, [c16])
            nd2 = jnp.maximum(n2r + n2c - 2.0 * d2, 0.0)
            pv = _tanh_pos(1.0 / (_sqrt16(nd2) + _EPS))
            pp = _tanh_pos(jnp.abs(d1))
            phia[pl.ds(off, _L)] = pp * pv

        pltpu.async_copy(phia.at[pl.ds(k * _CH, _CH)],
                         s_acc.at[rows.at[b]], ssems[b], add=True)
        pltpu.async_copy(onesb, deg_acc.at[rows.at[b]], ssems[b], add=True)

    fire_gathers(0, 0)

    def pair(i, carry):
        k0 = i * 2
        wait_gathers(0)
        fire_gathers(k0 + 1, 1)

        @pl.when(i > 0)
        def _dr0():
            drain_scatters(0)

        compute(k0, 0)
        wait_gathers(1)
        fire_gathers(k0 + 2, 0)

        @pl.when(i > 0)
        def _dr1():
            drain_scatters(1)

        compute(k0 + 1, 1)
        return carry

    lax.fori_loop(0, (_NCHUNK - 1) // 2, pair, 0)

    wait_gathers(0)
    drain_scatters(0)
    compute(_NCHUNK - 1, 0)
    drain_scatters(1)
    drain_scatters(0)

    pltpu.sync_copy(phia, phi_out.at[pl.ds(ebase, _EPW)])
    plsc.subcore_barrier()

    @pl.when(sid == 0)
    def _writeout():
        pltpu.sync_copy(s_acc, s_out.at[cid])
        pltpu.sync_copy(deg_acc, deg_out.at[cid])


_sc_phi = functools.partial(
    pl.kernel,
    mesh=plsc.VectorSubcoreMesh(core_axis_name="c", subcore_axis_name="s"),
    compiler_params=pltpu.CompilerParams(needs_layout_passes=False),
    out_type=[
        jax.ShapeDtypeStruct((_E,), jnp.float32),
        jax.ShapeDtypeStruct((_NC, _N), jnp.float32),
        jax.ShapeDtypeStruct((_NC, _N), jnp.float32),
    ],
    scratch_types=[
        pltpu.VMEM((_EPW,), jnp.int32),
        pltpu.VMEM((_EPW,), jnp.int32),
        pltpu.VMEM((2, _CH, _C), jnp.int32),
        pltpu.VMEM((2, _CH, _C), jnp.int32),
        pltpu.VMEM((_CH * _L,), jnp.float32),
        pltpu.VMEM((_CH * _L,), jnp.float32),
        pltpu.VMEM((_EPW,), jnp.float32),
        pltpu.VMEM((2, _CH), jnp.int32),
        pltpu.VMEM((_CH,), jnp.float32),
        pltpu.VMEM((_N,), jnp.float32),
        pltpu.VMEM_SHARED((_N,), jnp.float32),
        pltpu.VMEM_SHARED((_N,), jnp.float32),
        pltpu.SemaphoreType.DMA,
        pltpu.SemaphoreType.DMA,
        pltpu.SemaphoreType.DMA,
        pltpu.SemaphoreType.DMA,
    ],
)(_sc_phi_body)


_HN = _N // _NC
_JUNK = _HN
_TROWS = _HN + 8


def _sc_t_body(row_h, col_h, phi_h, x_h, zero2_h,
               t_out,
               rowa, cola, phia, adjv, xc, t_acc,
               gsem0, gsem1, tsem0, tsem1):
    cid = lax.axis_index("c")
    sid = lax.axis_index("s")
    ebase = sid * _EPT

    pltpu.sync_copy(row_h.at[pl.ds(ebase, _EPT)], rowa)
    pltpu.sync_copy(col_h.at[pl.ds(ebase, _EPT)], cola)
    pltpu.sync_copy(phi_h.at[pl.ds(ebase, _EPT)], phia)

    @pl.when(sid == 0)
    def _zero():
        pltpu.sync_copy(zero2_h, t_acc)

    plsc.subcore_barrier()

    row_lo = cid * _HN
    gsems = (gsem0, gsem1)
    tsems = (tsem0, tsem1)

    def fire_gather(k, b):
        pltpu.async_copy(x_h.at[cola.at[pl.ds(k * _CH2, _CH2)]],
                         xc.at[b], gsems[b])

    def wait_gather(b):
        pltpu.make_async_copy(x_h.at[cola.at[pl.ds(0, _CH2)]],
                              xc.at[b], gsems[b]).wait()

    def drain_scatter(b):
        pltpu.make_async_copy(xc.at[b], t_acc.at[adjv.at[b]],
                              tsems[b]).wait()

    def compute(k, b):
        for g in range(_CH2 // _L):
            rv = rowa[pl.ds(k * _CH2 + g * _L, _L)]
            adj = rv - row_lo
            ok = jnp.logical_and(adj >= 0, adj < _HN)
            adjv[b, pl.ds(g * _L, _L)] = jnp.where(ok, adj, _JUNK)

        def scale_e(e, c2):
            p = plsc.load_gather(
                phia, [jnp.full((_L,), k * _CH2 + e, jnp.int32)])
            for j in range(_C // _L):
                sl = pl.ds(_L * j, _L)
                xc[b, e, sl] = xc[b, e, sl] * p
            return c2

        lax.fori_loop(0, _CH2, scale_e, 0)

        pltpu.async_copy(xc.at[b], t_acc.at[adjv.at[b]], tsems[b], add=True)

    fire_gather(0, 0)

    def pair(i, carry):
        k0 = i * 2
        wait_gather(0)

        @pl.when(i > 0)
        def _dr1():
            drain_scatter(1)

        fire_gather(k0 + 1, 1)
        compute(k0, 0)
        wait_gather(1)
        drain_scatter(0)
        fire_gather(k0 + 2, 0)
        compute(k0 + 1, 1)
        return carry

    lax.fori_loop(0, (_NCHUNK2 - 2) // 2, pair, 0)

    wait_gather(0)
    drain_scatter(1)
    fire_gather(_NCHUNK2 - 1, 1)
    compute(_NCHUNK2 - 2, 0)
    wait_gather(1)
    drain_scatter(0)
    compute(_NCHUNK2 - 1, 1)
    drain_scatter(1)

    plsc.subcore_barrier()

    @pl.when(sid < 5)
    def _writeout():
        pltpu.sync_copy(t_acc.at[pl.ds(sid * 1000, 1000)],
                        t_out.at[cid, pl.ds(sid * 1000, 1000)])


_sc_t = functools.partial(
    pl.kernel,
    mesh=plsc.VectorSubcoreMesh(core_axis_name="c", subcore_axis_name="s"),
    compiler_params=pltpu.CompilerParams(needs_layout_passes=False),
    out_type=[
        jax.ShapeDtypeStruct((_NC, _HN, _C), jnp.float32),
    ],
    scratch_types=[
        pltpu.VMEM((_EPT,), jnp.int32),
        pltpu.VMEM((_EPT,), jnp.int32),
        pltpu.VMEM((_EPT,), jnp.float32),
        pltpu.VMEM((2, _CH2), jnp.int32),
        pltpu.VMEM((2, _CH2, _C), jnp.float32),
        pltpu.VMEM_SHARED((_TROWS, _C), jnp.float32),
        pltpu.SemaphoreType.DMA,
        pltpu.SemaphoreType.DMA,
        pltpu.SemaphoreType.DMA,
        pltpu.SemaphoreType.DMA,
    ],
)(_sc_t_body)


def _comb_body(x_ref, pc_ref, s_ref, dg_ref, t_ref, o_ref):
    xv = x_ref[...]
    s = s_ref[0] + s_ref[1]
    dg = dg_ref[0] + dg_ref[1]
    tt = t_ref[0]
    pc = pc_ref[...]
    scale = jnp.where(dg > 0, 0.5 / (dg * pc + 1e-30), 0.0)
    o_ref[...] = xv - scale * (s * xv - tt)


def _combine(x, phichi, s_parts, deg_parts, t_parts):
    n, c = x.shape
    blk = 1000
    return pl.pallas_call(
        _comb_body,
        grid=(n // blk,),
        in_specs=[
            pl.BlockSpec((blk, c), lambda i: (i, 0)),
            pl.BlockSpec((blk, 1), lambda i: (i, 0)),
            pl.BlockSpec((_NC, blk, 1), lambda i: (0, i, 0)),
            pl.BlockSpec((_NC, blk, 1), lambda i: (0, i, 0)),
            pl.BlockSpec((1, blk, _C), lambda i: (i // 5, i % 5, 0)),
        ],
        out_specs=pl.BlockSpec((blk, c), lambda i: (i, 0)),
        out_shape=jax.ShapeDtypeStruct((n, c), jnp.float32),
    )(x, phichi, s_parts, deg_parts, t_parts)


def kernel(x, edge_index, W_chi, W_phi, W_varphi):
    ab, phichi, n2 = _prep(x, W_chi, W_phi, W_varphi)
    row = edge_index[0]
    col = edge_index[1]
    zeros2 = jnp.zeros((_TROWS, _C), jnp.float32)
    zeros1 = jnp.zeros((_N,), jnp.float32)
    ab_i32 = jax.lax.bitcast_convert_type(
        ab.reshape(_N, _C, 2), jnp.int32)
    phi, s_parts, deg_parts = _sc_phi(row, col, ab_i32, n2[:, 0], zeros1)
    (t_parts,) = _sc_t(row, col, phi, x, zeros2)
    return _combine(x, phichi, s_parts[..., None], deg_parts[..., None],
                    t_parts)


# speedup vs baseline: 2.2893x; 1.2420x over previous
import functools
import jax
import jax.numpy as jnp
from jax import lax
from jax.experimental import pallas as pl
from jax.experimental.pallas import tpu as pltpu
from jax.experimental.pallas import tpu_sc as plsc

_N = 10000
_C = 128
_HC = _C // 2
_E = 320000
_NC = 2
_NS = 16
_NW = _NC * _NS
_EPW = _E // _NW
_CH = 80
_NCHUNK = _EPW // _CH
_EPT = _E // _NS
_CH2 = 80
_NCHUNK2 = _EPT // _CH2
_L = 16
_EPS = 1e-6


def _prep_body(x_ref, wc_ref, wp_ref, wv_ref, ab_ref, phichi_ref, n2_ref):
    xb = x_ref[...]
    y = jnp.dot(xb, wc_ref[...].T, preferred_element_type=jnp.float32)
    a = jnp.dot(y, wp_ref[...].T, preferred_element_type=jnp.float32)
    b = jnp.dot(xb, wv_ref[...].T, preferred_element_type=jnp.float32)
    ab_ref[:, :_C] = a.astype(jnp.bfloat16)
    ab_ref[:, _C:] = b.astype(jnp.bfloat16)
    phichi_ref[...] = jnp.tanh(jnp.sqrt(jnp.sum(y * y, axis=1, keepdims=True)))
    n2_ref[...] = jnp.sum(b * b, axis=1, keepdims=True)


def _prep(x, W_chi, W_phi, W_varphi):
    n, c = x.shape
    blk = 1000
    return pl.pallas_call(
        _prep_body,
        grid=(n // blk,),
        in_specs=[
            pl.BlockSpec((blk, c), lambda i: (i, 0)),
            pl.BlockSpec((c, c), lambda i: (0, 0)),
            pl.BlockSpec((c, c), lambda i: (0, 0)),
            pl.BlockSpec((c, c), lambda i: (0, 0)),
        ],
        out_specs=[
            pl.BlockSpec((blk, 2 * c), lambda i: (i, 0)),
            pl.BlockSpec((blk, 1), lambda i: (i, 0)),
            pl.BlockSpec((blk, 1), lambda i: (i, 0)),
        ],
        out_shape=[
            jax.ShapeDtypeStruct((n, 2 * c), jnp.bfloat16),
            jax.ShapeDtypeStruct((n, 1), jnp.float32),
            jax.ShapeDtypeStruct((n, 1), jnp.float32),
        ],
    )(x, W_chi, W_phi, W_varphi)


def _tanh_pos(t):
    t = jnp.minimum(t, 15.0)
    return 1.0 - 2.0 / (jnp.exp(2.0 * t) + 1.0)


def _sqrt16(v):
    i = plsc.bitcast(v, jnp.int32)
    r = plsc.bitcast(jnp.int32(0x5F3759DF) - (i >> 1), jnp.float32)
    for _ in range(3):
        r = r * (1.5 - 0.5 * v * r * r)
    return v * r


def _sc_phi_body(row_h, col_h, ab_h, n2_h, zero1_h,
                 phi_out, s_out, deg_out,
                 rowa, cola, abr, abc, acc1, acc2, phia, rows, onesb, n2v,
                 s_acc, deg_acc, gsem0, gsem1, ssem0, ssem1):
    cid = lax.axis_index("c")
    sid = lax.axis_index("s")
    wid = cid * _NS + sid
    ebase = wid * _EPW

    pltpu.sync_copy(n2_h, n2v)
    pltpu.sync_copy(row_h.at[pl.ds(ebase, _EPW)], rowa)
    pltpu.sync_copy(col_h.at[pl.ds(ebase, _EPW)], cola)
    for g in range(_CH // _L):
        onesb[pl.ds(g * _L, _L)] = jnp.full((_L,), 1.0, jnp.float32)

    @pl.when(sid == 0)
    def _zero():
        pltpu.sync_copy(zero1_h, s_acc)
        pltpu.sync_copy(zero1_h, deg_acc)

    plsc.subcore_barrier()

    gsems = (gsem0, gsem1)
    ssems = (ssem0, ssem1)

    def fire_gathers(k, b):
        sl = pl.ds(k * _CH, _CH)
        pltpu.async_copy(ab_h.at[rowa.at[sl]], abr.at[b], gsems[b])
        pltpu.async_copy(ab_h.at[cola.at[sl]], abc.at[b], gsems[b])

    def wait_gathers(b):
        sl = pl.ds(0, _CH)
        pltpu.make_async_copy(ab_h.at[rowa.at[sl]], abr.at[b],
                              gsems[b]).wait()
        pltpu.make_async_copy(ab_h.at[cola.at[sl]], abc.at[b],
                              gsems[b]).wait()

    def drain_scatters(b):
        pltpu.make_async_copy(phia.at[pl.ds(0, _CH)],
                              s_acc.at[rows.at[b]], ssems[b]).wait()
        pltpu.make_async_copy(onesb, deg_acc.at[rows.at[b]],
                              ssems[b]).wait()

    def compute(k, b):
        @plsc.parallel_loop(0, _CH, step=1, unroll=2)
        def dot_e(e):
            def dot_half(off):
                acc = None
                for j in range(_C // (2 * _L)):
                    ar = plsc.bitcast(abr[b, e, pl.ds(off + _L * j, _L)],
                                      jnp.bfloat16)
                    ac = plsc.bitcast(abc[b, e, pl.ds(off + _L * j, _L)],
                                      jnp.bfloat16)
                    term = ar * ac
                    acc = term if acc is None else acc + term
                u1, u2 = plsc.unpack(
                    acc, format=plsc.PackFormat.INTERLEAVED,
                    preferred_element_type=jnp.float32)
                return u1 + u2

            acc1[pl.ds(e * _L, _L)] = dot_half(0)
            acc2[pl.ds(e * _L, _L)] = dot_half(_C // 2)

        for g in range(_CH // _L):
            fl = (lax.iota(jnp.int32, _L) + (g * _L)) * _L
            d1 = plsc.load_gather(acc1, [fl])
            d2 = plsc.load_gather(acc2, [fl])
            for j in range(1, _L):
                d1 = d1 + plsc.load_gather(acc1, [fl + j])
                d2 = d2 + plsc.load_gather(acc2, [fl + j])
            off = k * _CH + g * _L
            r16 = rowa[pl.ds(off, _L)]
            c16 = cola[pl.ds(off, _L)]
            rows[b, pl.ds(g * _L, _L)] = r16
            n2r = plsc.load_gather(n2v, [r16])
            n2c = plsc.load_gather(n2v, [c16])
            nd2 = jnp.maximum(n2r + n2c - 2.0 * d2, 0.0)
            pv = _tanh_pos(1.0 / (_sqrt16(nd2) + _EPS))
            pp = _tanh_pos(jnp.abs(d1))
            phia[pl.ds(off, _L)] = pp * pv

        pltpu.async_copy(phia.at[pl.ds(k * _CH, _CH)],
                         s_acc.at[rows.at[b]], ssems[b], add=True)
        pltpu.async_copy(onesb, deg_acc.at[rows.at[b]], ssems[b], add=True)

    fire_gathers(0, 0)

    def pair(i, carry):
        k0 = i * 2
        wait_gathers(0)
        fire_gathers(k0 + 1, 1)

        @pl.when(i > 0)
        def _dr0():
            drain_scatters(0)

        compute(k0, 0)
        wait_gathers(1)
        fire_gathers(k0 + 2, 0)

        @pl.when(i > 0)
        def _dr1():
            drain_scatters(1)

        compute(k0 + 1, 1)
        return carry

    lax.fori_loop(0, (_NCHUNK - 1) // 2, pair, 0)

    wait_gathers(0)
    drain_scatters(0)
    compute(_NCHUNK - 1, 0)
    drain_scatters(1)
    drain_scatters(0)

    pltpu.sync_copy(phia, phi_out.at[pl.ds(ebase, _EPW)])
    plsc.subcore_barrier()

    @pl.when(sid == 0)
    def _writeout():
        pltpu.sync_copy(s_acc, s_out.at[cid])
        pltpu.sync_copy(deg_acc, deg_out.at[cid])


_sc_phi = functools.partial(
    pl.kernel,
    mesh=plsc.VectorSubcoreMesh(core_axis_name="c", subcore_axis_name="s"),
    compiler_params=pltpu.CompilerParams(needs_layout_passes=False),
    out_type=[
        jax.ShapeDtypeStruct((_E,), jnp.float32),
        jax.ShapeDtypeStruct((_NC, _N), jnp.float32),
        jax.ShapeDtypeStruct((_NC, _N), jnp.float32),
    ],
    scratch_types=[
        pltpu.VMEM((_EPW,), jnp.int32),
        pltpu.VMEM((_EPW,), jnp.int32),
        pltpu.VMEM((2, _CH, _C), jnp.int32),
        pltpu.VMEM((2, _CH, _C), jnp.int32),
        pltpu.VMEM((_CH * _L,), jnp.float32),
        pltpu.VMEM((_CH * _L,), jnp.float32),
        pltpu.VMEM((_EPW,), jnp.float32),
        pltpu.VMEM((2, _CH), jnp.int32),
        pltpu.VMEM((_CH,), jnp.float32),
        pltpu.VMEM((_N,), jnp.float32),
        pltpu.VMEM_SHARED((_N,), jnp.float32),
        pltpu.VMEM_SHARED((_N,), jnp.float32),
        pltpu.SemaphoreType.DMA,
        pltpu.SemaphoreType.DMA,
        pltpu.SemaphoreType.DMA,
        pltpu.SemaphoreType.DMA,
    ],
)(_sc_phi_body)


_HN = _N // _NC
_JUNK = _HN
_TROWS = _HN + 8


def _sc_t_body(row_h, col_h, phi_h, x_h, zero2_h,
               t_out,
               rowa, cola, phia, adjv, xc, t_acc,
               gsem0, gsem1, tsem0, tsem1):
    cid = lax.axis_index("c")
    sid = lax.axis_index("s")
    ebase = sid * _EPT

    pltpu.sync_copy(row_h.at[pl.ds(ebase, _EPT)], rowa)
    pltpu.sync_copy(col_h.at[pl.ds(ebase, _EPT)], cola)
    pltpu.sync_copy(phi_h.at[pl.ds(ebase, _EPT)], phia)

    @pl.when(sid == 0)
    def _zero():
        pltpu.sync_copy(zero2_h, t_acc)

    plsc.subcore_barrier()

    row_lo = cid * _HN
    gsems = (gsem0, gsem1)
    tsems = (tsem0, tsem1)

    def fire_gather(k, b):
        pltpu.async_copy(x_h.at[cola.at[pl.ds(k * _CH2, _CH2)]],
                         xc.at[b], gsems[b])

    def wait_gather(b):
        pltpu.make_async_copy(x_h.at[cola.at[pl.ds(0, _CH2)]],
                              xc.at[b], gsems[b]).wait()

    def drain_scatter(b):
        pltpu.make_async_copy(xc.at[b], t_acc.at[adjv.at[b]],
                              tsems[b]).wait()

    def compute(k, b):
        for g in range(_CH2 // _L):
            rv = rowa[pl.ds(k * _CH2 + g * _L, _L)]
            adj = rv - row_lo
            ok = jnp.logical_and(adj >= 0, adj < _HN)
            adjv[b, pl.ds(g * _L, _L)] = jnp.where(ok, adj, _JUNK)

        @plsc.parallel_loop(0, _CH2, step=1, unroll=2)
        def scale_e(e):
            p = plsc.load_gather(
                phia, [jnp.full((_L,), k * _CH2 + e, jnp.int32)])
            for j in range(_C // _L):
                sl = pl.ds(_L * j, _L)
                xc[b, e, sl] = xc[b, e, sl] * p

        pltpu.async_copy(xc.at[b], t_acc.at[adjv.at[b]], tsems[b], add=True)

    fire_gather(0, 0)

    def pair(i, carry):
        k0 = i * 2
        wait_gather(0)

        @pl.when(i > 0)
        def _dr1():
            drain_scatter(1)

        fire_gather(k0 + 1, 1)
        compute(k0, 0)
        wait_gather(1)
        drain_scatter(0)
        fire_gather(k0 + 2, 0)
        compute(k0 + 1, 1)
        return carry

    lax.fori_loop(0, (_NCHUNK2 - 2) // 2, pair, 0)

    wait_gather(0)
    drain_scatter(1)
    fire_gather(_NCHUNK2 - 1, 1)
    compute(_NCHUNK2 - 2, 0)
    wait_gather(1)
    drain_scatter(0)
    compute(_NCHUNK2 - 1, 1)
    drain_scatter(1)

    plsc.subcore_barrier()

    @pl.when(sid < 5)
    def _writeout():
        pltpu.sync_copy(t_acc.at[pl.ds(sid * 1000, 1000)],
                        t_out.at[cid, pl.ds(sid * 1000, 1000)])


_sc_t = functools.partial(
    pl.kernel,
    mesh=plsc.VectorSubcoreMesh(core_axis_name="c", subcore_axis_name="s"),
    compiler_params=pltpu.CompilerParams(needs_layout_passes=False),
    out_type=[
        jax.ShapeDtypeStruct((_NC, _HN, _C), jnp.float32),
    ],
    scratch_types=[
        pltpu.VMEM((_EPT,), jnp.int32),
        pltpu.VMEM((_EPT,), jnp.int32),
        pltpu.VMEM((_EPT,), jnp.float32),
        pltpu.VMEM((2, _CH2), jnp.int32),
        pltpu.VMEM((2, _CH2, _C), jnp.float32),
        pltpu.VMEM_SHARED((_TROWS, _C), jnp.float32),
        pltpu.SemaphoreType.DMA,
        pltpu.SemaphoreType.DMA,
        pltpu.SemaphoreType.DMA,
        pltpu.SemaphoreType.DMA,
    ],
)(_sc_t_body)


def _comb_body(x_ref, pc_ref, s_ref, dg_ref, t_ref, o_ref):
    xv = x_ref[...]
    s = s_ref[0] + s_ref[1]
    dg = dg_ref[0] + dg_ref[1]
    tt = t_ref[0]
    pc = pc_ref[...]
    scale = jnp.where(dg > 0, 0.5 / (dg * pc + 1e-30), 0.0)
    o_ref[...] = xv - scale * (s * xv - tt)


def _combine(x, phichi, s_parts, deg_parts, t_parts):
    n, c = x.shape
    blk = 1000
    return pl.pallas_call(
        _comb_body,
        grid=(n // blk,),
        in_specs=[
            pl.BlockSpec((blk, c), lambda i: (i, 0)),
            pl.BlockSpec((blk, 1), lambda i: (i, 0)),
            pl.BlockSpec((_NC, blk, 1), lambda i: (0, i, 0)),
            pl.BlockSpec((_NC, blk, 1), lambda i: (0, i, 0)),
            pl.BlockSpec((1, blk, _C), lambda i: (i // 5, i % 5, 0)),
        ],
        out_specs=pl.BlockSpec((blk, c), lambda i: (i, 0)),
        out_shape=jax.ShapeDtypeStruct((n, c), jnp.float32),
    )(x, phichi, s_parts, deg_parts, t_parts)


def kernel(x, edge_index, W_chi, W_phi, W_varphi):
    ab, phichi, n2 = _prep(x, W_chi, W_phi, W_varphi)
    row = edge_index[0]
    col = edge_index[1]
    zeros2 = jnp.zeros((_TROWS, _C), jnp.float32)
    zeros1 = jnp.zeros((_N,), jnp.float32)
    ab_i32 = jax.lax.bitcast_convert_type(
        ab.reshape(_N, _C, 2), jnp.int32)
    phi, s_parts, deg_parts = _sc_phi(row, col, ab_i32, n2[:, 0], zeros1)
    (t_parts,) = _sc_t(row, col, phi, x, zeros2)
    return _combine(x, phichi, s_parts[..., None], deg_parts[..., None],
                    t_parts)
